# baseline (speedup 1.0000x reference)
import jax
import jax.numpy as jnp
from jax import lax
from jax.experimental import pallas as pl
from jax.experimental.pallas import tpu as pltpu
from jax.experimental.pallas import tpu_sc as plsc

N_NODES = 10000
N_EDGES = 320000
DIM = 128

NC = 2
NS = 16
NW = NC * NS
CHUNK = 128
NCHUNK = 80
EPAD = NW * NCHUNK * CHUNK - N_EDGES
NPAD = 10240
RPT = NPAD // NS


def _matmul_body(x_ref, wfc_ref, wa_ref, z_ref, st_ref):
    xb = x_ref[...]
    zb = lax.dot_general(xb, wfc_ref[...], (((1,), (1,)), ((), ())),
                         preferred_element_type=jnp.float32)
    z_ref[...] = zb
    st_ref[...] = lax.dot_general(wa_ref[...], zb, (((1,), (1,)), ((), ())),
                                  preferred_element_type=jnp.float32)


def _combine_body(acc_ref, den_ref, out_ref):
    a = acc_ref[0, :N_NODES] + acc_ref[1, :N_NODES]
    d = den_ref[0, :N_NODES] + den_ref[1, :N_NODES]
    d = jnp.where(d == 0.0, 1.0, d)
    out_ref[...] = a / d[:, None]


def _sc_body(st_hbm, src_hbm, dst_hbm, z_hbm, acc_out, den_out,
             rows_a, rows_b, w_a, w_b, sch_a, sch_b, tch_a, tch_b,
             se_a, de_a, se_b, de_b,
             s_sp, t_sp, acc_sp, den_sp,
             sem_sa, sem_sb):
    c = lax.axis_index("c")
    s = lax.axis_index("s")
    wid = c * NS + s
    base = wid * NCHUNK

    zero16 = jnp.zeros((16,), jnp.float32)

    def issue_s(rows, w, de, sem):
        pltpu.async_copy(rows, acc_sp.at[de], sem, add=True)
        pltpu.async_copy(w.at[pl.ds(0, CHUNK)], den_sp.at[de], sem, add=True)

    def wait_s(rows, w, de, sem):
        pltpu.make_async_copy(rows, acc_sp.at[de], sem).wait()
        pltpu.make_async_copy(w.at[pl.ds(0, CHUNK)], den_sp.at[de], sem).wait()

    def fetch_e(j, se, de):
        pltpu.sync_copy(src_hbm.at[j], se)
        pltpu.sync_copy(dst_hbm.at[j], de)

    def gather(se, de, rows, sch, tch):
        pltpu.sync_copy(z_hbm.at[se], rows)
        pltpu.sync_copy(s_sp.at[se], sch)
        pltpu.sync_copy(t_sp.at[de], tch)

    def compute(rows, w, sch, tch):
        for k in range(CHUNK // 16):
            e = sch[pl.ds(k * 16, 16)] + tch[pl.ds(k * 16, 16)]
            e = jnp.where(e >= 0.0, e, 0.01 * e)
            w[pl.ds(k * 16, 16)] = jnp.exp(e)

        def _srow(b, carry):
            wb = w[pl.ds(b, 16)][0]
            for q in range(DIM // 16):
                rows[b, pl.ds(q * 16, 16)] = rows[b, pl.ds(q * 16, 16)] * wb
            return carry
        lax.fori_loop(0, CHUNK, _srow, 0, unroll=4)

    def _zrow_a(r, carry):
        for q in range(DIM // 16):
            rows_a[r, pl.ds(q * 16, 16)] = zero16
        return carry
    lax.fori_loop(0, CHUNK, _zrow_a, 0)

    def _zrow_b(r, carry):
        for q in range(DIM // 16):
            rows_b[r, pl.ds(q * 16, 16)] = zero16
        return carry
    lax.fori_loop(0, CHUNK, _zrow_b, 0)
    for q in range((CHUNK + 16) // 16):
        w_a[pl.ds(q * 16, 16)] = zero16
        w_b[pl.ds(q * 16, 16)] = zero16

    for i in range(RPT // CHUNK):
        pltpu.sync_copy(rows_a, acc_sp.at[pl.ds(s * RPT + i * CHUNK, CHUNK)])
    for i in range(RPT // CHUNK):
        pltpu.sync_copy(rows_a.at[0], den_sp.at[pl.ds(s * RPT + i * CHUNK, CHUNK)])

    @pl.when(s == 0)
    def _stage():
        pltpu.sync_copy(st_hbm.at[0], s_sp)
        pltpu.sync_copy(st_hbm.at[1], t_sp)

    plsc.subcore_barrier()

    fetch_e(base + 0, se_a, de_a)
    fetch_e(base + 1, se_b, de_b)
    issue_s(rows_a, w_a, de_a, sem_sa)
    issue_s(rows_b, w_b, de_b, sem_sb)

    def _pair(k, carry):
        ja = base + 2 * k
        wait_s(rows_a, w_a, de_a, sem_sa)
        fetch_e(ja, se_a, de_a)
        gather(se_a, de_a, rows_a, sch_a, tch_a)
        compute(rows_a, w_a, sch_a, tch_a)
        issue_s(rows_a, w_a, de_a, sem_sa)
        wait_s(rows_b, w_b, de_b, sem_sb)
        fetch_e(ja + 1, se_b, de_b)
        gather(se_b, de_b, rows_b, sch_b, tch_b)
        compute(rows_b, w_b, sch_b, tch_b)
        issue_s(rows_b, w_b, de_b, sem_sb)
        return carry
    lax.fori_loop(0, NCHUNK // 2, _pair, 0)

    wait_s(rows_a, w_a, de_a, sem_sa)
    wait_s(rows_b, w_b, de_b, sem_sb)

    plsc.subcore_barrier()

    pltpu.sync_copy(acc_sp.at[pl.ds(s * RPT, RPT)],
                    acc_out.at[c, pl.ds(s * RPT, RPT)])
    pltpu.sync_copy(den_sp.at[pl.ds(s * RPT, RPT)],
                    den_out.at[pl.ds(c * NPAD + s * RPT, RPT)])


@jax.jit
def kernel(x, edge_index, W_fc, W_attn):
    x = x.astype(jnp.float32)
    W_fc = W_fc.astype(jnp.float32)
    W_attn = W_attn.astype(jnp.float32)
    wa = jnp.zeros((8, DIM), jnp.float32)
    wa = wa.at[0].set(W_attn[0, :DIM]).at[1].set(W_attn[0, DIM:])

    pad_src = (jnp.arange(EPAD, dtype=jnp.int32) * 7) % N_NODES
    pad_dst = N_NODES + (jnp.arange(EPAD, dtype=jnp.int32) % (NPAD - N_NODES))
    src = jnp.concatenate([edge_index[0].astype(jnp.int32), pad_src])
    dst = jnp.concatenate([edge_index[1].astype(jnp.int32), pad_dst])
    src = src.reshape(NW * NCHUNK, CHUNK)
    dst = dst.reshape(NW * NCHUNK, CHUNK)

    z, st = pl.pallas_call(
        _matmul_body,
        out_shape=[
            jax.ShapeDtypeStruct((N_NODES, DIM), jnp.float32),
            jax.ShapeDtypeStruct((8, N_NODES), jnp.float32),
        ],
    )(x, W_fc, wa)

    sc = pl.kernel(
        _sc_body,
        out_type=[
            jax.ShapeDtypeStruct((NC, NPAD, DIM), jnp.float32),
            jax.ShapeDtypeStruct((NC * NPAD,), jnp.float32),
        ],
        mesh=plsc.VectorSubcoreMesh(core_axis_name="c", subcore_axis_name="s"),
        compiler_params=pltpu.CompilerParams(needs_layout_passes=False),
        scratch_types=[
            pltpu.VMEM((CHUNK, DIM), jnp.float32),
            pltpu.VMEM((CHUNK, DIM), jnp.float32),
            pltpu.VMEM((CHUNK + 16,), jnp.float32),
            pltpu.VMEM((CHUNK + 16,), jnp.float32),
            pltpu.VMEM((CHUNK,), jnp.float32),
            pltpu.VMEM((CHUNK,), jnp.float32),
            pltpu.VMEM((CHUNK,), jnp.float32),
            pltpu.VMEM((CHUNK,), jnp.float32),
            pltpu.VMEM((CHUNK,), jnp.int32),
            pltpu.VMEM((CHUNK,), jnp.int32),
            pltpu.VMEM((CHUNK,), jnp.int32),
            pltpu.VMEM((CHUNK,), jnp.int32),
            pltpu.VMEM_SHARED((N_NODES,), jnp.float32),
            pltpu.VMEM_SHARED((N_NODES,), jnp.float32),
            pltpu.VMEM_SHARED((NPAD, DIM), jnp.float32),
            pltpu.VMEM_SHARED((NPAD,), jnp.float32),
            pltpu.SemaphoreType.DMA,
            pltpu.SemaphoreType.DMA,
        ],
    )
    acc, den = sc(st, src, dst, z)

    den = den.reshape(NC, NPAD)

    h = pl.pallas_call(
        _combine_body,
        out_shape=jax.ShapeDtypeStruct((N_NODES, DIM), jnp.float32),
    )(acc, den)
    return h


# speedup vs baseline: 1.1893x; 1.1893x over previous
import jax
import jax.numpy as jnp
from jax import lax
from jax.experimental import pallas as pl
from jax.experimental.pallas import tpu as pltpu
from jax.experimental.pallas import tpu_sc as plsc

N_NODES = 10000
N_EDGES = 320000
DIM = 128

NC = 2
NS = 16
NW = NC * NS
CHUNK = 128
NCHUNK = 79
EPAD = NW * NCHUNK * CHUNK - N_EDGES
NPAD = 10240
RPT = NPAD // NS


def _matmul_body(x_ref, wfc_ref, wa_ref, z_ref, st_ref):
    xb = x_ref[...]
    zb = lax.dot_general(xb, wfc_ref[...], (((1,), (1,)), ((), ())),
                         preferred_element_type=jnp.float32)
    z_ref[...] = zb
    st_ref[...] = lax.dot_general(wa_ref[...], zb, (((1,), (1,)), ((), ())),
                                  preferred_element_type=jnp.float32)


def _combine_body(acc_ref, den_ref, out_ref):
    a = acc_ref[0, :N_NODES] + acc_ref[1, :N_NODES]
    d = den_ref[0, :N_NODES] + den_ref[1, :N_NODES]
    d = jnp.where(d == 0.0, 1.0, d)
    out_ref[...] = a / d[:, None]


def _sc_body(st_hbm, src_hbm, dst_hbm, z_hbm, acc_out, den_out,
             src_v, dst_v, rows_v, w_v, sch_v, tch_v,
             s_sp, t_sp, acc_sp, den_sp):
    c = lax.axis_index("c")
    s = lax.axis_index("s")
    wid = c * NS + s

    zero16 = jnp.zeros((16,), jnp.float32)

    def _zrow(r, carry):
        for q in range(DIM // 16):
            rows_v[r, pl.ds(q * 16, 16)] = zero16
        return carry
    lax.fori_loop(0, CHUNK, _zrow, 0)

    for i in range(RPT // CHUNK):
        pltpu.sync_copy(rows_v, acc_sp.at[pl.ds(s * RPT + i * CHUNK, CHUNK)])
    for i in range(RPT // CHUNK):
        pltpu.sync_copy(rows_v.at[0], den_sp.at[pl.ds(s * RPT + i * CHUNK, CHUNK)])

    @pl.when(s == 0)
    def _stage():
        pltpu.sync_copy(st_hbm.at[0], s_sp)
        pltpu.sync_copy(st_hbm.at[1], t_sp)
    pltpu.sync_copy(src_hbm.at[wid], src_v)
    pltpu.sync_copy(dst_hbm.at[wid], dst_v)

    plsc.subcore_barrier()

    def _chunk(j, carry):
        pltpu.sync_copy(z_hbm.at[src_v.at[j]], rows_v)
        pltpu.sync_copy(s_sp.at[src_v.at[j]], sch_v)
        pltpu.sync_copy(t_sp.at[dst_v.at[j]], tch_v)
        for k in range(CHUNK // 16):
            e = sch_v[pl.ds(k * 16, 16)] + tch_v[pl.ds(k * 16, 16)]
            e = jnp.where(e >= 0.0, e, 0.01 * e)
            w_v[pl.ds(k * 16, 16)] = jnp.exp(e)

        @plsc.parallel_loop(0, CHUNK, unroll=4)
        def _srow(b):
            wb = w_v[pl.ds(b, 16)][0]
            for q in range(DIM // 16):
                rows_v[b, pl.ds(q * 16, 16)] = rows_v[b, pl.ds(q * 16, 16)] * wb

        pltpu.sync_copy(rows_v, acc_sp.at[dst_v.at[j]], add=True)
        pltpu.sync_copy(w_v.at[pl.ds(0, CHUNK)], den_sp.at[dst_v.at[j]], add=True)
        return carry
    lax.fori_loop(0, NCHUNK, _chunk, 0)

    plsc.subcore_barrier()

    pltpu.sync_copy(acc_sp.at[pl.ds(s * RPT, RPT)],
                    acc_out.at[c, pl.ds(s * RPT, RPT)])
    pltpu.sync_copy(den_sp.at[pl.ds(s * RPT, RPT)],
                    den_out.at[pl.ds(c * NPAD + s * RPT, RPT)])


@jax.jit
def kernel(x, edge_index, W_fc, W_attn):
    x = x.astype(jnp.float32)
    W_fc = W_fc.astype(jnp.float32)
    W_attn = W_attn.astype(jnp.float32)
    wa = jnp.zeros((8, DIM), jnp.float32)
    wa = wa.at[0].set(W_attn[0, :DIM]).at[1].set(W_attn[0, DIM:])

    pad_src = (jnp.arange(EPAD, dtype=jnp.int32) * 7) % N_NODES
    pad_dst = N_NODES + (jnp.arange(EPAD, dtype=jnp.int32) % (NPAD - N_NODES))
    src = jnp.concatenate([edge_index[0].astype(jnp.int32), pad_src])
    dst = jnp.concatenate([edge_index[1].astype(jnp.int32), pad_dst])
    src = src.reshape(NW, NCHUNK, CHUNK)
    dst = dst.reshape(NW, NCHUNK, CHUNK)

    z, st = pl.pallas_call(
        _matmul_body,
        out_shape=[
            jax.ShapeDtypeStruct((N_NODES, DIM), jnp.float32),
            jax.ShapeDtypeStruct((8, N_NODES), jnp.float32),
        ],
    )(x, W_fc, wa)

    sc = pl.kernel(
        _sc_body,
        out_type=[
            jax.ShapeDtypeStruct((NC, NPAD, DIM), jnp.float32),
            jax.ShapeDtypeStruct((NC * NPAD,), jnp.float32),
        ],
        mesh=plsc.VectorSubcoreMesh(core_axis_name="c", subcore_axis_name="s"),
        compiler_params=pltpu.CompilerParams(needs_layout_passes=False),
        scratch_types=[
            pltpu.VMEM((NCHUNK, CHUNK), jnp.int32),
            pltpu.VMEM((NCHUNK, CHUNK), jnp.int32),
            pltpu.VMEM((CHUNK, DIM), jnp.float32),
            pltpu.VMEM((CHUNK + 16,), jnp.float32),
            pltpu.VMEM((CHUNK,), jnp.float32),
            pltpu.VMEM((CHUNK,), jnp.float32),
            pltpu.VMEM_SHARED((N_NODES,), jnp.float32),
            pltpu.VMEM_SHARED((N_NODES,), jnp.float32),
            pltpu.VMEM_SHARED((NPAD, DIM), jnp.float32),
            pltpu.VMEM_SHARED((NPAD,), jnp.float32),
        ],
    )
    acc, den = sc(st, src, dst, z)

    den = den.reshape(NC, NPAD)

    h = pl.pallas_call(
        _combine_body,
        out_shape=jax.ShapeDtypeStruct((N_NODES, DIM), jnp.float32),
    )(acc, den)
    return h


# speedup vs baseline: 1.2083x; 1.0160x over previous
import jax
import jax.numpy as jnp
from jax import lax
from jax.experimental import pallas as pl
from jax.experimental.pallas import tpu as pltpu
from jax.experimental.pallas import tpu_sc as plsc

N_NODES = 10000
N_EDGES = 320000
DIM = 128

NC = 2
NS = 16
NW = NC * NS
CHUNK = 128
NCHUNK = 79
EPAD = NW * NCHUNK * CHUNK - N_EDGES
NPAD = 10240
RPT = NPAD // NS


def _matmul_body(x_ref, wfc_ref, wa_ref, z_ref, st_ref):
    xb = x_ref[...]
    zb = lax.dot_general(xb, wfc_ref[...], (((1,), (1,)), ((), ())),
                         preferred_element_type=jnp.float32)
    z_ref[...] = zb
    st_ref[...] = lax.dot_general(wa_ref[...], zb, (((1,), (1,)), ((), ())),
                                  preferred_element_type=jnp.float32)


def _combine_body(acc_ref, den_ref, out_ref):
    a = acc_ref[0, :N_NODES] + acc_ref[1, :N_NODES]
    d = den_ref[0, :N_NODES] + den_ref[1, :N_NODES]
    d = jnp.where(d == 0.0, 1.0, d)
    out_ref[...] = a / d[:, None]


def _sc_body(st_hbm, src_hbm, dst_hbm, z_hbm, acc_out, den_out,
             src_v, dst_v, rows_v, w_v, sch_v, tch_v,
             s_sp, t_sp, acc_sp, den_sp, sem_d):
    c = lax.axis_index("c")
    s = lax.axis_index("s")
    wid = c * NS + s

    zero16 = jnp.zeros((16,), jnp.float32)

    def _zrow(r, carry):
        for q in range(DIM // 16):
            rows_v[r, pl.ds(q * 16, 16)] = zero16
        return carry
    lax.fori_loop(0, CHUNK, _zrow, 0)

    for i in range(RPT // CHUNK):
        pltpu.sync_copy(rows_v, acc_sp.at[pl.ds(s * RPT + i * CHUNK, CHUNK)])
    for i in range(RPT // CHUNK):
        pltpu.sync_copy(rows_v.at[0], den_sp.at[pl.ds(s * RPT + i * CHUNK, CHUNK)])

    @pl.when(s == 0)
    def _stage():
        pltpu.sync_copy(st_hbm.at[0], s_sp)
        pltpu.sync_copy(st_hbm.at[1], t_sp)
    pltpu.sync_copy(src_hbm.at[wid], src_v)
    pltpu.sync_copy(dst_hbm.at[wid], dst_v)

    plsc.subcore_barrier()

    def _chunk(j, carry):
        pltpu.sync_copy(z_hbm.at[src_v.at[j]], rows_v)
        pltpu.sync_copy(s_sp.at[src_v.at[j]], sch_v)
        pltpu.sync_copy(t_sp.at[dst_v.at[j]], tch_v)
        for k in range(CHUNK // 16):
            e = sch_v[pl.ds(k * 16, 16)] + tch_v[pl.ds(k * 16, 16)]
            e = jnp.where(e >= 0.0, e, 0.01 * e)
            w_v[pl.ds(k * 16, 16)] = jnp.exp(e)

        @plsc.parallel_loop(0, CHUNK, unroll=8)
        def _srow(b):
            wb = w_v[pl.ds(b, 16)][0]
            for q in range(DIM // 16):
                rows_v[b, pl.ds(q * 16, 16)] = rows_v[b, pl.ds(q * 16, 16)] * wb

        den_copy = pltpu.async_copy(
            w_v.at[pl.ds(0, CHUNK)], den_sp.at[dst_v.at[j]], sem_d, add=True)
        pltpu.sync_copy(rows_v, acc_sp.at[dst_v.at[j]], add=True)
        den_copy.wait()
        return carry
    lax.fori_loop(0, NCHUNK, _chunk, 0)

    plsc.subcore_barrier()

    pltpu.sync_copy(acc_sp.at[pl.ds(s * RPT, RPT)],
                    acc_out.at[c, pl.ds(s * RPT, RPT)])
    pltpu.sync_copy(den_sp.at[pl.ds(s * RPT, RPT)],
                    den_out.at[pl.ds(c * NPAD + s * RPT, RPT)])


@jax.jit
def kernel(x, edge_index, W_fc, W_attn):
    x = x.astype(jnp.float32)
    W_fc = W_fc.astype(jnp.float32)
    W_attn = W_attn.astype(jnp.float32)
    wa = jnp.zeros((8, DIM), jnp.float32)
    wa = wa.at[0].set(W_attn[0, :DIM]).at[1].set(W_attn[0, DIM:])

    pad_src = (jnp.arange(EPAD, dtype=jnp.int32) * 7) % N_NODES
    pad_dst = N_NODES + (jnp.arange(EPAD, dtype=jnp.int32) % (NPAD - N_NODES))
    src = jnp.concatenate([edge_index[0].astype(jnp.int32), pad_src])
    dst = jnp.concatenate([edge_index[1].astype(jnp.int32), pad_dst])
    src = src.reshape(NW, NCHUNK, CHUNK)
    dst = dst.reshape(NW, NCHUNK, CHUNK)

    z, st = pl.pallas_call(
        _matmul_body,
        out_shape=[
            jax.ShapeDtypeStruct((N_NODES, DIM), jnp.float32),
            jax.ShapeDtypeStruct((8, N_NODES), jnp.float32),
        ],
    )(x, W_fc, wa)

    sc = pl.kernel(
        _sc_body,
        out_type=[
            jax.ShapeDtypeStruct((NC, NPAD, DIM), jnp.float32),
            jax.ShapeDtypeStruct((NC * NPAD,), jnp.float32),
        ],
        mesh=plsc.VectorSubcoreMesh(core_axis_name="c", subcore_axis_name="s"),
        compiler_params=pltpu.CompilerParams(needs_layout_passes=False),
        scratch_types=[
            pltpu.VMEM((NCHUNK, CHUNK), jnp.int32),
            pltpu.VMEM((NCHUNK, CHUNK), jnp.int32),
            pltpu.VMEM((CHUNK, DIM), jnp.float32),
            pltpu.VMEM((CHUNK + 16,), jnp.float32),
            pltpu.VMEM((CHUNK,), jnp.float32),
            pltpu.VMEM((CHUNK,), jnp.float32),
            pltpu.VMEM_SHARED((N_NODES,), jnp.float32),
            pltpu.VMEM_SHARED((N_NODES,), jnp.float32),
            pltpu.VMEM_SHARED((NPAD, DIM), jnp.float32),
            pltpu.VMEM_SHARED((NPAD,), jnp.float32),
            pltpu.SemaphoreType.DMA,
        ],
    )
    acc, den = sc(st, src, dst, z)

    den = den.reshape(NC, NPAD)

    h = pl.pallas_call(
        _combine_body,
        out_shape=jax.ShapeDtypeStruct((N_NODES, DIM), jnp.float32),
    )(acc, den)
    return h


# speedup vs baseline: 1.2896x; 1.0672x over previous
import jax
import jax.numpy as jnp
from jax import lax
from jax.experimental import pallas as pl
from jax.experimental.pallas import tpu as pltpu
from jax.experimental.pallas import tpu_sc as plsc

N_NODES = 10000
N_EDGES = 320000
DIM = 128

NC = 2
NS = 16
NW = NC * NS
CHUNK = 128
NCHUNK = 80
EPAD = NW * NCHUNK * CHUNK - N_EDGES
NPAD = 10240
RPT = NPAD // NS


def _matmul_body(x_ref, wfc_ref, wa_ref, z_ref, st_ref):
    xb = x_ref[...]
    zb = lax.dot_general(xb, wfc_ref[...], (((1,), (1,)), ((), ())),
                         preferred_element_type=jnp.float32)
    z_ref[...] = zb
    st_ref[...] = lax.dot_general(wa_ref[...], zb, (((1,), (1,)), ((), ())),
                                  preferred_element_type=jnp.float32)


def _combine_body(acc_ref, den_ref, out_ref):
    a = acc_ref[0, :N_NODES] + acc_ref[1, :N_NODES]
    d = den_ref[0, :N_NODES] + den_ref[1, :N_NODES]
    d = jnp.where(d == 0.0, 1.0, d)
    out_ref[...] = a / d[:, None]


def _sc_body(st_hbm, src_hbm, dst_hbm, z_hbm, acc_out, den_out,
             src_v, dst_v, rows_v, w_a, w_b, sch_v, tch_v,
             s_sp, t_sp, acc_sp, den_sp, sem_sa, sem_sb):
    c = lax.axis_index("c")
    s = lax.axis_index("s")
    wid = c * NS + s

    zero16 = jnp.zeros((16,), jnp.float32)

    def _zrow(r, carry):
        for q in range(DIM // 16):
            rows_v[r, pl.ds(q * 16, 16)] = zero16
        return carry
    lax.fori_loop(0, CHUNK, _zrow, 0)

    for i in range(RPT // CHUNK):
        pltpu.sync_copy(rows_v, acc_sp.at[pl.ds(s * RPT + i * CHUNK, CHUNK)])
    for i in range(RPT // CHUNK):
        pltpu.sync_copy(rows_v.at[0], den_sp.at[pl.ds(s * RPT + i * CHUNK, CHUNK)])

    @pl.when(s == 0)
    def _stage():
        pltpu.sync_copy(st_hbm.at[0], s_sp)
        pltpu.sync_copy(st_hbm.at[1], t_sp)
    pltpu.sync_copy(src_hbm.at[wid], src_v)
    pltpu.sync_copy(dst_hbm.at[wid], dst_v)
    for q in range((CHUNK + 16) // 16):
        w_a[pl.ds(q * 16, 16)] = zero16
        w_b[pl.ds(q * 16, 16)] = zero16

    plsc.subcore_barrier()

    def issue_s(w, de, sem):
        a = pltpu.async_copy(rows_v, acc_sp.at[de], sem, add=True)
        d = pltpu.async_copy(w.at[pl.ds(0, CHUNK)], den_sp.at[de], sem, add=True)
        return a, d

    def wait_s(w, de, sem):
        pltpu.make_async_copy(rows_v, acc_sp.at[de], sem).wait()
        pltpu.make_async_copy(w.at[pl.ds(0, CHUNK)], den_sp.at[de], sem).wait()

    def front(j, w):
        pltpu.sync_copy(s_sp.at[src_v.at[j]], sch_v)
        pltpu.sync_copy(t_sp.at[dst_v.at[j]], tch_v)
        for k in range(CHUNK // 16):
            e = sch_v[pl.ds(k * 16, 16)] + tch_v[pl.ds(k * 16, 16)]
            e = jnp.where(e >= 0.0, e, 0.01 * e)
            w[pl.ds(k * 16, 16)] = jnp.exp(e)

    def back(j, w, sem):
        pltpu.sync_copy(z_hbm.at[src_v.at[j]], rows_v)

        @plsc.parallel_loop(0, CHUNK, unroll=8)
        def _srow(b):
            wb = w[pl.ds(b, 16)][0]
            for q in range(DIM // 16):
                rows_v[b, pl.ds(q * 16, 16)] = rows_v[b, pl.ds(q * 16, 16)] * wb

        return issue_s(w, dst_v.at[j], sem)

    issue_s(w_b, dst_v.at[0], sem_sb)

    def _pair(k, carry):
        ja = 2 * k
        jm_b = jnp.maximum(ja - 1, 0)
        front(ja, w_a)
        wait_s(w_b, dst_v.at[jm_b], sem_sb)
        sa, da = back(ja, w_a, sem_sa)
        front(ja + 1, w_b)
        sa.wait()
        da.wait()
        issue_b = back(ja + 1, w_b, sem_sb)
        del issue_b
        return carry
    lax.fori_loop(0, NCHUNK // 2, _pair, 0)

    wait_s(w_b, dst_v.at[NCHUNK - 1], sem_sb)

    plsc.subcore_barrier()

    pltpu.sync_copy(acc_sp.at[pl.ds(s * RPT, RPT)],
                    acc_out.at[c, pl.ds(s * RPT, RPT)])
    pltpu.sync_copy(den_sp.at[pl.ds(s * RPT, RPT)],
                    den_out.at[pl.ds(c * NPAD + s * RPT, RPT)])


@jax.jit
def kernel(x, edge_index, W_fc, W_attn):
    x = x.astype(jnp.float32)
    W_fc = W_fc.astype(jnp.float32)
    W_attn = W_attn.astype(jnp.float32)
    wa = jnp.zeros((8, DIM), jnp.float32)
    wa = wa.at[0].set(W_attn[0, :DIM]).at[1].set(W_attn[0, DIM:])

    pad_src = (jnp.arange(EPAD, dtype=jnp.int32) * 7) % N_NODES
    pad_dst = N_NODES + (jnp.arange(EPAD, dtype=jnp.int32) % (NPAD - N_NODES))
    src = jnp.concatenate([edge_index[0].astype(jnp.int32), pad_src])
    dst = jnp.concatenate([edge_index[1].astype(jnp.int32), pad_dst])
    src = src.reshape(NW, NCHUNK, CHUNK)
    dst = dst.reshape(NW, NCHUNK, CHUNK)

    z, st = pl.pallas_call(
        _matmul_body,
        out_shape=[
            jax.ShapeDtypeStruct((N_NODES, DIM), jnp.float32),
            jax.ShapeDtypeStruct((8, N_NODES), jnp.float32),
        ],
    )(x, W_fc, wa)

    sc = pl.kernel(
        _sc_body,
        out_type=[
            jax.ShapeDtypeStruct((NC, NPAD, DIM), jnp.float32),
            jax.ShapeDtypeStruct((NC * NPAD,), jnp.float32),
        ],
        mesh=plsc.VectorSubcoreMesh(core_axis_name="c", subcore_axis_name="s"),
        compiler_params=pltpu.CompilerParams(needs_layout_passes=False),
        scratch_types=[
            pltpu.VMEM((NCHUNK, CHUNK), jnp.int32),
            pltpu.VMEM((NCHUNK, CHUNK), jnp.int32),
            pltpu.VMEM((CHUNK, DIM), jnp.float32),
            pltpu.VMEM((CHUNK + 16,), jnp.float32),
            pltpu.VMEM((CHUNK + 16,), jnp.float32),
            pltpu.VMEM((CHUNK,), jnp.float32),
            pltpu.VMEM((CHUNK,), jnp.float32),
            pltpu.VMEM_SHARED((N_NODES,), jnp.float32),
            pltpu.VMEM_SHARED((N_NODES,), jnp.float32),
            pltpu.VMEM_SHARED((NPAD, DIM), jnp.float32),
            pltpu.VMEM_SHARED((NPAD,), jnp.float32),
            pltpu.SemaphoreType.DMA,
            pltpu.SemaphoreType.DMA,
        ],
    )
    acc, den = sc(st, src, dst, z)

    den = den.reshape(NC, NPAD)

    h = pl.pallas_call(
        _combine_body,
        out_shape=jax.ShapeDtypeStruct((N_NODES, DIM), jnp.float32),
    )(acc, den)
    return h
